# baseline (speedup 1.0000x reference)
import functools

import jax
import jax.numpy as jnp
from jax import lax
from jax.experimental import pallas as pl
from jax.experimental.pallas import tpu as pltpu
from jax.experimental.pallas import tpu_sc as plsc

_N = 10000
_DIM = 128
_HALF = 64
_E = 320000
_B = 16384
_NTILES = 16
_NCORES = 2
_CH = 128
_ER = _E // _CH
_ERPT = _ER // _NTILES
_ERTAIL = _ER - _ERPT * _NTILES
_RCH = 80
_NRB = _N // _RCH
_RBPT = -(-_NRB // _NTILES)
_ACC_ROWS = _N + 16
_BPW = _B // (_NTILES * _NCORES)


def _maxnorm_tc_kernel(x_ref, o_ref):
    x = x_ref[...]
    ss = jnp.sum(x * x, axis=1, keepdims=True)
    n = jnp.sqrt(ss)
    scale = jnp.minimum(1.0, 1.0 / jnp.maximum(n, 1e-7))
    y = x * scale
    o_ref[0] = y[:, :_HALF]
    o_ref[1] = y[:, _HALF:]


def _maxnorm_split(entity_table):
    blk = 1000
    out = pl.pallas_call(
        _maxnorm_tc_kernel,
        grid=(_N // blk,),
        in_specs=[pl.BlockSpec((blk, _DIM), lambda g: (g, 0))],
        out_specs=pl.BlockSpec((2, blk, _HALF), lambda g: (0, g, 0)),
        out_shape=jax.ShapeDtypeStruct((2, _N, _HALF), jnp.float32),
    )(entity_table)
    return out.reshape(2 * _N, _HALF)


_SC_MESH = plsc.VectorSubcoreMesh(core_axis_name="c", subcore_axis_name="s")
_SC_PARAMS = pltpu.CompilerParams(use_tc_tiling_on_sc=False,
                                  needs_layout_passes=False)


@functools.partial(
    pl.kernel,
    out_type=jax.ShapeDtypeStruct((2 * _N, _HALF), jnp.float32),
    mesh=_SC_MESH,
    scratch_types=[
        pltpu.VMEM_SHARED((_ACC_ROWS, _HALF), jnp.float32),
        pltpu.VMEM_SHARED((_ACC_ROWS, _HALF), jnp.float32),
        pltpu.VMEM((_ERPT + 1, _CH), jnp.int32),
        pltpu.VMEM((1, _CH), jnp.int32),
        pltpu.VMEM((1, _CH), jnp.int32),
        pltpu.VMEM((_CH, _HALF), jnp.float32),
        pltpu.VMEM((_CH, _HALF), jnp.float32),
        pltpu.VMEM((_CH, _HALF), jnp.float32),
        pltpu.SemaphoreType.DMA,
        pltpu.SemaphoreType.DMA,
        pltpu.SemaphoreType.DMA,
        pltpu.SemaphoreType.DMA,
    ],
    compiler_params=_SC_PARAMS,
)
def _k2_aggregate(x_hbm, src_hbm, dst_hbm, h_hbm, acc_s, acc_q, srcb,
                  dstb0, dstb1, rows_a, rows_b, sq,
                  sem_g0, sem_g1, sem_i0, sem_i1):
    c = lax.axis_index("c")
    s = lax.axis_index("s")
    coff = c * _N

    e0 = s * _ERPT
    pltpu.sync_copy(src_hbm.at[pl.ds(e0, _ERPT)], srcb.at[pl.ds(0, _ERPT)])

    @pl.when(s < _ERTAIL)
    def _():
        et = _NTILES * _ERPT + s
        pltpu.sync_copy(src_hbm.at[pl.ds(et, 1)], srcb.at[pl.ds(_ERPT, 1)])

    @pl.loop(0, _ERPT + 1)
    def _(r):
        for k in range(_CH // 16):
            sl = pl.ds(k * 16, 16)
            srcb[r, sl] = srcb[r, sl] + coff

    @pl.loop(0, _RCH)
    def _(r):
        for k in range(_HALF // 16):
            rows_a[r, pl.ds(k * 16, 16)] = jnp.zeros((16,), jnp.float32)

    for j in range(_RBPT):
        blk = s + j * _NTILES
        @pl.when(blk < _NRB)
        def _():
            r0 = pl.multiple_of(blk * _RCH, 8)
            pltpu.sync_copy(rows_a.at[pl.ds(0, _RCH)], acc_s.at[pl.ds(r0, _RCH)])
            pltpu.sync_copy(rows_a.at[pl.ds(0, _RCH)], acc_q.at[pl.ds(r0, _RCH)])
    plsc.subcore_barrier()

    def _fire_g(j, rb, sem):
        pltpu.async_copy(x_hbm.at[srcb.at[j]], rb, sem)

    def _drain_g(rb, sem):
        pltpu.make_async_copy(x_hbm.at[srcb.at[0]], rb, sem).wait()

    def _fire_i(j, db, sem):
        pltpu.async_copy(dst_hbm.at[pl.ds(e0 + j, 1)], db, sem)

    def _drain_i(db, sem):
        pltpu.make_async_copy(dst_hbm.at[pl.ds(0, 1)], db, sem).wait()

    def _work(rb, db):
        @pl.loop(0, _CH, unroll=4)
        def _(r):
            for k in range(_HALF // 16):
                sl = pl.ds(k * 16, 16)
                v = rb[r, sl]
                sq[r, sl] = v * v
        pltpu.sync_copy(rb, acc_s.at[db.at[0]], add=True)
        pltpu.sync_copy(sq, acc_q.at[db.at[0]], add=True)

    _fire_i(0, dstb0, sem_i0)
    _fire_i(1, dstb1, sem_i1)
    _fire_g(0, rows_a, sem_g0)

    nhalf = _ERPT // 2

    @pl.loop(0, nhalf)
    def _(jj):
        j0 = 2 * jj
        _fire_g(j0 + 1, rows_b, sem_g1)
        _drain_g(rows_a, sem_g0)
        _drain_i(dstb0, sem_i0)
        _work(rows_a, dstb0)

        @pl.when(jj < nhalf - 1)
        def _():
            _fire_i(j0 + 2, dstb0, sem_i0)
            _fire_g(j0 + 2, rows_a, sem_g0)
        _drain_g(rows_b, sem_g1)
        _drain_i(dstb1, sem_i1)
        _work(rows_b, dstb1)

        @pl.when(jj < nhalf - 1)
        def _():
            _fire_i(j0 + 3, dstb1, sem_i1)

    @pl.when(s < _ERTAIL)
    def _():
        _fire_i(_NTILES * _ERPT + s - e0, dstb0, sem_i0)
        _fire_g(_ERPT, rows_a, sem_g0)
        _drain_g(rows_a, sem_g0)
        _drain_i(dstb0, sem_i0)
        _work(rows_a, dstb0)

    plsc.subcore_barrier()

    for j in range(_RBPT):
        blk = s + j * _NTILES
        @pl.when(blk < _NRB)
        def _():
            r0 = pl.multiple_of(blk * _RCH, 8)
            g0 = pl.multiple_of(coff + r0, 8)
            pltpu.sync_copy(acc_s.at[pl.ds(r0, _RCH)], rows_a.at[pl.ds(0, _RCH)])
            pltpu.sync_copy(acc_q.at[pl.ds(r0, _RCH)], sq.at[pl.ds(0, _RCH)])
            pltpu.sync_copy(x_hbm.at[pl.ds(g0, _RCH)], rows_b.at[pl.ds(0, _RCH)])

            @pl.loop(0, _RCH)
            def _(r):
                for k in range(_HALF // 16):
                    sl = pl.ds(k * 16, 16)
                    sm = rows_a[r, sl]
                    rows_a[r, sl] = sm * sm - sq[r, sl] + rows_b[r, sl]

            pltpu.sync_copy(rows_a.at[pl.ds(0, _RCH)], h_hbm.at[pl.ds(g0, _RCH)])


@functools.partial(
    pl.kernel,
    out_type=jax.ShapeDtypeStruct((_B,), jnp.float32),
    mesh=_SC_MESH,
    scratch_types=[
        pltpu.VMEM((2, _CH), jnp.int32),
        pltpu.VMEM((2, _CH), jnp.int32),
        pltpu.VMEM((2, _CH), jnp.int32),
        pltpu.VMEM((2, _CH, _DIM), jnp.float32),
        pltpu.VMEM((2, _CH, _HALF), jnp.float32),
        pltpu.VMEM((2, _CH, _HALF), jnp.float32),
        pltpu.VMEM((_CH,), jnp.float32),
        pltpu.SemaphoreType.DMA,
        pltpu.SemaphoreType.DMA,
    ],
    compiler_params=_SC_PARAMS,
)
def _k3_predict(ut_hbm, h_hbm, u_hbm, i_hbm, out_hbm, ub, ib, ib2, ur, il, ih,
                ob, sem_a, sem_b):
    c = lax.axis_index("c")
    s = lax.axis_index("s")
    wid = s * _NCORES + c
    lanes = lax.broadcasted_iota(jnp.int32, (16,), 0)
    sems = (sem_a, sem_b)
    nch = _BPW // _CH

    def _fetch(ch):
        p = ch % 2
        base = pl.multiple_of(wid * _BPW + ch * _CH, 8)
        pltpu.sync_copy(u_hbm.at[pl.ds(base, _CH)], ub.at[p])
        pltpu.sync_copy(i_hbm.at[pl.ds(base, _CH)], ib.at[p])
        for k in range(_CH // 16):
            sl = pl.ds(k * 16, 16)
            ib2[p, sl] = ib[p, sl] + _N
        pltpu.async_copy(ut_hbm.at[ub.at[p]], ur.at[p], sems[p])
        pltpu.async_copy(h_hbm.at[ib.at[p]], il.at[p], sems[p])
        pltpu.async_copy(h_hbm.at[ib2.at[p]], ih.at[p], sems[p])

    def _drain(ch):
        p = ch % 2
        pltpu.make_async_copy(ut_hbm.at[ub.at[p]], ur.at[p], sems[p]).wait()
        pltpu.make_async_copy(h_hbm.at[ib.at[p]], il.at[p], sems[p]).wait()
        pltpu.make_async_copy(h_hbm.at[ib.at[p]], ih.at[p], sems[p]).wait()

    _fetch(0)
    for ch in range(nch):
        if ch + 1 < nch:
            _fetch(ch + 1)
        _drain(ch)
        p = ch % 2
        base = pl.multiple_of(wid * _BPW + ch * _CH, 8)

        @pl.loop(0, _CH // 16)
        def _(g):
            dot_v = jnp.zeros((16,), jnp.float32)
            ss_v = jnp.zeros((16,), jnp.float32)
            for t in range(16):
                b = g * 16 + t
                acc = jnp.zeros((16,), jnp.float32)
                sacc = jnp.zeros((16,), jnp.float32)
                for k in range(_DIM // 16):
                    uvk = ur[p, b, pl.ds(k * 16, 16)]
                    if k < _HALF // 16:
                        ivk = il[p, b, pl.ds(k * 16, 16)]
                    else:
                        ivk = ih[p, b, pl.ds((k - _HALF // 16) * 16, 16)]
                    acc = acc + uvk * ivk
                    sacc = sacc + uvk * uvk
                dot_v = jnp.where(lanes == t, jnp.sum(acc), dot_v)
                ss_v = jnp.where(lanes == t, jnp.sum(sacc), ss_v)
            y = plsc.bitcast(jnp.int32(0x5F3759DF) - (plsc.bitcast(ss_v, jnp.int32) >> 1),
                             jnp.float32)
            for _ in range(3):
                y = y * (1.5 - 0.5 * ss_v * y * y)
            uvdot = dot_v * jnp.minimum(1.0, y)
            ob[pl.ds(g * 16, 16)] = 1.0 / (1.0 + jnp.exp(-uvdot))

        pltpu.sync_copy(ob, out_hbm.at[pl.ds(base, _CH)])


def kernel(user_table, entity_table, u, i, edge_index):
    u = u.astype(jnp.int32)
    i = i.astype(jnp.int32)
    src = edge_index[0].astype(jnp.int32).reshape(_ER, _CH)
    dst = edge_index[1].astype(jnp.int32).reshape(_ER, _CH)
    x_cat = _maxnorm_split(entity_table)
    h_cat = _k2_aggregate(x_cat, src, dst)
    return _k3_predict(user_table, h_cat, u, i)


# speedup vs baseline: 2.1727x; 2.1727x over previous
import functools

import jax
import jax.numpy as jnp
from jax import lax
from jax.experimental import pallas as pl
from jax.experimental.pallas import tpu as pltpu
from jax.experimental.pallas import tpu_sc as plsc

_N = 10000
_DIM = 128
_HALF = 64
_E = 320000
_B = 16384
_NTILES = 16
_NCORES = 2
_CH = 128
_ER = _E // _CH
_ERPT = _ER // _NTILES
_ERTAIL = _ER - _ERPT * _NTILES
_RCH = 80
_NRB = _N // _RCH
_RBPT = -(-_NRB // _NTILES)
_ACC_ROWS = _N + 16
_BPW = _B // (_NTILES * _NCORES)


def _maxnorm_tc_kernel(x_ref, o_ref):
    x = x_ref[...]
    ss = jnp.sum(x * x, axis=1, keepdims=True)
    n = jnp.sqrt(ss)
    scale = jnp.minimum(1.0, 1.0 / jnp.maximum(n, 1e-7))
    y = x * scale
    o_ref[0] = y[:, :_HALF]
    o_ref[1] = y[:, _HALF:]


def _maxnorm_split(entity_table):
    blk = 1000
    out = pl.pallas_call(
        _maxnorm_tc_kernel,
        grid=(_N // blk,),
        in_specs=[pl.BlockSpec((blk, _DIM), lambda g: (g, 0))],
        out_specs=pl.BlockSpec((2, blk, _HALF), lambda g: (0, g, 0)),
        out_shape=jax.ShapeDtypeStruct((2, _N, _HALF), jnp.float32),
    )(entity_table)
    return out.reshape(2 * _N, _HALF)


_SC_MESH = plsc.VectorSubcoreMesh(core_axis_name="c", subcore_axis_name="s")
_SC_PARAMS = pltpu.CompilerParams(use_tc_tiling_on_sc=False,
                                  needs_layout_passes=False)


@functools.partial(
    pl.kernel,
    out_type=jax.ShapeDtypeStruct((2 * _N, _HALF), jnp.float32),
    mesh=_SC_MESH,
    scratch_types=[
        pltpu.VMEM_SHARED((_ACC_ROWS, _HALF), jnp.float32),
        pltpu.VMEM_SHARED((_ACC_ROWS, _HALF), jnp.float32),
        pltpu.VMEM((_ERPT + 1, _CH), jnp.int32),
        pltpu.VMEM((1, _CH), jnp.int32),
        pltpu.VMEM((1, _CH), jnp.int32),
        pltpu.VMEM((_CH, _HALF), jnp.float32),
        pltpu.VMEM((_CH, _HALF), jnp.float32),
        pltpu.VMEM((_CH, _HALF), jnp.float32),
        pltpu.SemaphoreType.DMA,
        pltpu.SemaphoreType.DMA,
        pltpu.SemaphoreType.DMA,
        pltpu.SemaphoreType.DMA,
    ],
    compiler_params=_SC_PARAMS,
)
def _k2_aggregate(x_hbm, src_hbm, dst_hbm, h_hbm, acc_s, acc_q, srcb,
                  dstb0, dstb1, rows_a, rows_b, sq,
                  sem_g0, sem_g1, sem_i0, sem_i1):
    c = lax.axis_index("c")
    s = lax.axis_index("s")
    coff = c * _N

    e0 = s * _ERPT
    pltpu.sync_copy(src_hbm.at[pl.ds(e0, _ERPT)], srcb.at[pl.ds(0, _ERPT)])

    @pl.when(s < _ERTAIL)
    def _():
        et = _NTILES * _ERPT + s
        pltpu.sync_copy(src_hbm.at[pl.ds(et, 1)], srcb.at[pl.ds(_ERPT, 1)])

    @pl.loop(0, _ERPT + 1)
    def _(r):
        for k in range(_CH // 16):
            sl = pl.ds(k * 16, 16)
            srcb[r, sl] = srcb[r, sl] + coff

    @pl.loop(0, _RCH)
    def _(r):
        for k in range(_HALF // 16):
            rows_a[r, pl.ds(k * 16, 16)] = jnp.zeros((16,), jnp.float32)

    for j in range(_RBPT):
        blk = s + j * _NTILES
        @pl.when(blk < _NRB)
        def _():
            r0 = pl.multiple_of(blk * _RCH, 8)
            pltpu.sync_copy(rows_a.at[pl.ds(0, _RCH)], acc_s.at[pl.ds(r0, _RCH)])
            pltpu.sync_copy(rows_a.at[pl.ds(0, _RCH)], acc_q.at[pl.ds(r0, _RCH)])
    plsc.subcore_barrier()

    def _fire_g(j, rb, sem):
        pltpu.async_copy(x_hbm.at[srcb.at[j]], rb, sem)

    def _drain_g(rb, sem):
        pltpu.make_async_copy(x_hbm.at[srcb.at[0]], rb, sem).wait()

    def _fire_i(j, db, sem):
        pltpu.async_copy(dst_hbm.at[pl.ds(e0 + j, 1)], db, sem)

    def _drain_i(db, sem):
        pltpu.make_async_copy(dst_hbm.at[pl.ds(0, 1)], db, sem).wait()

    def _work(rb, db):
        @pl.loop(0, _CH // 4)
        def _(r4):
            r0 = r4 * 4
            for dr in range(4):
                for k in range(_HALF // 16):
                    sl = pl.ds(k * 16, 16)
                    v = rb[r0 + dr, sl]
                    sq[r0 + dr, sl] = v * v
        pltpu.sync_copy(rb, acc_s.at[db.at[0]], add=True)
        pltpu.sync_copy(sq, acc_q.at[db.at[0]], add=True)

    _fire_i(0, dstb0, sem_i0)
    _fire_i(1, dstb1, sem_i1)
    _fire_g(0, rows_a, sem_g0)

    nhalf = _ERPT // 2

    @pl.loop(0, nhalf)
    def _(jj):
        j0 = 2 * jj
        _fire_g(j0 + 1, rows_b, sem_g1)
        _drain_g(rows_a, sem_g0)
        _drain_i(dstb0, sem_i0)
        _work(rows_a, dstb0)

        @pl.when(jj < nhalf - 1)
        def _():
            _fire_i(j0 + 2, dstb0, sem_i0)
            _fire_g(j0 + 2, rows_a, sem_g0)
        _drain_g(rows_b, sem_g1)
        _drain_i(dstb1, sem_i1)
        _work(rows_b, dstb1)

        @pl.when(jj < nhalf - 1)
        def _():
            _fire_i(j0 + 3, dstb1, sem_i1)

    @pl.when(s < _ERTAIL)
    def _():
        _fire_i(_NTILES * _ERPT + s - e0, dstb0, sem_i0)
        _fire_g(_ERPT, rows_a, sem_g0)
        _drain_g(rows_a, sem_g0)
        _drain_i(dstb0, sem_i0)
        _work(rows_a, dstb0)

    plsc.subcore_barrier()

    for j in range(_RBPT):
        blk = s + j * _NTILES
        @pl.when(blk < _NRB)
        def _():
            r0 = pl.multiple_of(blk * _RCH, 8)
            g0 = pl.multiple_of(coff + r0, 8)
            pltpu.sync_copy(acc_s.at[pl.ds(r0, _RCH)], rows_a.at[pl.ds(0, _RCH)])
            pltpu.sync_copy(acc_q.at[pl.ds(r0, _RCH)], sq.at[pl.ds(0, _RCH)])
            pltpu.sync_copy(x_hbm.at[pl.ds(g0, _RCH)], rows_b.at[pl.ds(0, _RCH)])

            @pl.loop(0, _RCH)
            def _(r):
                for k in range(_HALF // 16):
                    sl = pl.ds(k * 16, 16)
                    sm = rows_a[r, sl]
                    rows_a[r, sl] = sm * sm - sq[r, sl] + rows_b[r, sl]

            pltpu.sync_copy(rows_a.at[pl.ds(0, _RCH)], h_hbm.at[pl.ds(g0, _RCH)])


@functools.partial(
    pl.kernel,
    out_type=jax.ShapeDtypeStruct((_B,), jnp.float32),
    mesh=_SC_MESH,
    scratch_types=[
        pltpu.VMEM((2, _CH), jnp.int32),
        pltpu.VMEM((2, _CH), jnp.int32),
        pltpu.VMEM((2, _CH), jnp.int32),
        pltpu.VMEM((2, _CH, _DIM), jnp.float32),
        pltpu.VMEM((2, _CH, _HALF), jnp.float32),
        pltpu.VMEM((2, _CH, _HALF), jnp.float32),
        pltpu.VMEM((_CH,), jnp.float32),
        pltpu.SemaphoreType.DMA,
        pltpu.SemaphoreType.DMA,
    ],
    compiler_params=_SC_PARAMS,
)
def _k3_predict(ut_hbm, h_hbm, u_hbm, i_hbm, out_hbm, ub, ib, ib2, ur, il, ih,
                ob, sem_a, sem_b):
    c = lax.axis_index("c")
    s = lax.axis_index("s")
    wid = s * _NCORES + c
    lanes = lax.broadcasted_iota(jnp.int32, (16,), 0)
    sems = (sem_a, sem_b)
    nch = _BPW // _CH

    def _fetch(ch):
        p = ch % 2
        base = pl.multiple_of(wid * _BPW + ch * _CH, 8)
        pltpu.sync_copy(u_hbm.at[pl.ds(base, _CH)], ub.at[p])
        pltpu.sync_copy(i_hbm.at[pl.ds(base, _CH)], ib.at[p])
        for k in range(_CH // 16):
            sl = pl.ds(k * 16, 16)
            ib2[p, sl] = ib[p, sl] + _N
        pltpu.async_copy(ut_hbm.at[ub.at[p]], ur.at[p], sems[p])
        pltpu.async_copy(h_hbm.at[ib.at[p]], il.at[p], sems[p])
        pltpu.async_copy(h_hbm.at[ib2.at[p]], ih.at[p], sems[p])

    def _drain(ch):
        p = ch % 2
        pltpu.make_async_copy(ut_hbm.at[ub.at[p]], ur.at[p], sems[p]).wait()
        pltpu.make_async_copy(h_hbm.at[ib.at[p]], il.at[p], sems[p]).wait()
        pltpu.make_async_copy(h_hbm.at[ib.at[p]], ih.at[p], sems[p]).wait()

    _fetch(0)
    for ch in range(nch):
        if ch + 1 < nch:
            _fetch(ch + 1)
        _drain(ch)
        p = ch % 2
        base = pl.multiple_of(wid * _BPW + ch * _CH, 8)

        @pl.loop(0, _CH // 16)
        def _(g):
            dot_v = jnp.zeros((16,), jnp.float32)
            ss_v = jnp.zeros((16,), jnp.float32)
            for t in range(16):
                b = g * 16 + t
                acc = jnp.zeros((16,), jnp.float32)
                sacc = jnp.zeros((16,), jnp.float32)
                for k in range(_DIM // 16):
                    uvk = ur[p, b, pl.ds(k * 16, 16)]
                    if k < _HALF // 16:
                        ivk = il[p, b, pl.ds(k * 16, 16)]
                    else:
                        ivk = ih[p, b, pl.ds((k - _HALF // 16) * 16, 16)]
                    acc = acc + uvk * ivk
                    sacc = sacc + uvk * uvk
                dot_v = jnp.where(lanes == t, jnp.sum(acc), dot_v)
                ss_v = jnp.where(lanes == t, jnp.sum(sacc), ss_v)
            y = plsc.bitcast(jnp.int32(0x5F3759DF) - (plsc.bitcast(ss_v, jnp.int32) >> 1),
                             jnp.float32)
            for _ in range(3):
                y = y * (1.5 - 0.5 * ss_v * y * y)
            uvdot = dot_v * jnp.minimum(1.0, y)
            ob[pl.ds(g * 16, 16)] = 1.0 / (1.0 + jnp.exp(-uvdot))

        pltpu.sync_copy(ob, out_hbm.at[pl.ds(base, _CH)])


def kernel(user_table, entity_table, u, i, edge_index):
    u = u.astype(jnp.int32)
    i = i.astype(jnp.int32)
    src = edge_index[0].astype(jnp.int32).reshape(_ER, _CH)
    dst = edge_index[1].astype(jnp.int32).reshape(_ER, _CH)
    x_cat = _maxnorm_split(entity_table)
    h_cat = _k2_aggregate(x_cat, src, dst)
    return _k3_predict(user_table, h_cat, u, i)


# speedup vs baseline: 2.1886x; 1.0073x over previous
import functools

import jax
import jax.numpy as jnp
from jax import lax
from jax.experimental import pallas as pl
from jax.experimental.pallas import tpu as pltpu
from jax.experimental.pallas import tpu_sc as plsc

_N = 10000
_DIM = 128
_HALF = 64
_E = 320000
_B = 16384
_NTILES = 16
_NCORES = 2
_CH = 128
_ER = _E // _CH
_ERPT = _ER // _NTILES
_ERTAIL = _ER - _ERPT * _NTILES
_RCH = 80
_NRB = _N // _RCH
_RBPT = -(-_NRB // _NTILES)
_ACC_ROWS = _N + 16
_BPW = _B // (_NTILES * _NCORES)


def _maxnorm_tc_kernel(x_ref, o_ref):
    x = x_ref[...]
    ss = jnp.sum(x * x, axis=1, keepdims=True)
    n = jnp.sqrt(ss)
    scale = jnp.minimum(1.0, 1.0 / jnp.maximum(n, 1e-7))
    y = x * scale
    o_ref[0] = y[:, :_HALF]
    o_ref[1] = y[:, _HALF:]


def _maxnorm_split(entity_table):
    blk = 1000
    out = pl.pallas_call(
        _maxnorm_tc_kernel,
        grid=(_N // blk,),
        in_specs=[pl.BlockSpec((blk, _DIM), lambda g: (g, 0))],
        out_specs=pl.BlockSpec((2, blk, _HALF), lambda g: (0, g, 0)),
        out_shape=jax.ShapeDtypeStruct((2, _N, _HALF), jnp.float32),
    )(entity_table)
    return out.reshape(2 * _N, _HALF)


_SC_MESH = plsc.VectorSubcoreMesh(core_axis_name="c", subcore_axis_name="s")
_SC_PARAMS = pltpu.CompilerParams(use_tc_tiling_on_sc=False,
                                  needs_layout_passes=False)


@functools.partial(
    pl.kernel,
    out_type=jax.ShapeDtypeStruct((2 * _N, _HALF), jnp.float32),
    mesh=_SC_MESH,
    scratch_types=[
        pltpu.VMEM_SHARED((_ACC_ROWS, _HALF), jnp.float32),
        pltpu.VMEM_SHARED((_ACC_ROWS, _HALF), jnp.float32),
        pltpu.VMEM((_ERPT + 1, _CH), jnp.int32),
        pltpu.VMEM((1, _CH), jnp.int32),
        pltpu.VMEM((1, _CH), jnp.int32),
        pltpu.VMEM((_CH, _HALF), jnp.float32),
        pltpu.VMEM((_CH, _HALF), jnp.float32),
        pltpu.VMEM((_CH, _HALF), jnp.float32),
        pltpu.SemaphoreType.DMA,
        pltpu.SemaphoreType.DMA,
        pltpu.SemaphoreType.DMA,
        pltpu.SemaphoreType.DMA,
    ],
    compiler_params=_SC_PARAMS,
)
def _k2_aggregate(x_hbm, src_hbm, dst_hbm, h_hbm, acc_s, acc_q, srcb,
                  dstb0, dstb1, rows_a, rows_b, sq,
                  sem_g0, sem_g1, sem_i0, sem_i1):
    c = lax.axis_index("c")
    s = lax.axis_index("s")
    coff = c * _N

    e0 = s * _ERPT
    pltpu.sync_copy(src_hbm.at[pl.ds(e0, _ERPT)], srcb.at[pl.ds(0, _ERPT)])

    @pl.when(s < _ERTAIL)
    def _():
        et = _NTILES * _ERPT + s
        pltpu.sync_copy(src_hbm.at[pl.ds(et, 1)], srcb.at[pl.ds(_ERPT, 1)])

    @pl.loop(0, _ERPT + 1)
    def _(r):
        for k in range(_CH // 16):
            sl = pl.ds(k * 16, 16)
            srcb[r, sl] = srcb[r, sl] + coff

    @pl.loop(0, _RCH)
    def _(r):
        for k in range(_HALF // 16):
            rows_a[r, pl.ds(k * 16, 16)] = jnp.zeros((16,), jnp.float32)

    for j in range(_RBPT):
        blk = s + j * _NTILES
        @pl.when(blk < _NRB)
        def _():
            r0 = pl.multiple_of(blk * _RCH, 8)
            pltpu.sync_copy(rows_a.at[pl.ds(0, _RCH)], acc_s.at[pl.ds(r0, _RCH)])
            pltpu.sync_copy(rows_a.at[pl.ds(0, _RCH)], acc_q.at[pl.ds(r0, _RCH)])
    plsc.subcore_barrier()

    def _fire_g(j, rb, sem):
        pltpu.async_copy(x_hbm.at[srcb.at[j]], rb, sem)

    def _drain_g(rb, sem):
        pltpu.make_async_copy(x_hbm.at[srcb.at[0]], rb, sem).wait()

    def _fire_i(j, db, sem):
        pltpu.async_copy(dst_hbm.at[pl.ds(e0 + j, 1)], db, sem)

    def _drain_i(db, sem):
        pltpu.make_async_copy(dst_hbm.at[pl.ds(0, 1)], db, sem).wait()

    def _work(rb, db):
        @pl.loop(0, _CH // 8)
        def _(r8):
            r0 = r8 * 8
            for dr in range(8):
                for k in range(_HALF // 16):
                    sl = pl.ds(k * 16, 16)
                    v = rb[r0 + dr, sl]
                    sq[r0 + dr, sl] = v * v
        pltpu.sync_copy(rb, acc_s.at[db.at[0]], add=True)
        pltpu.sync_copy(sq, acc_q.at[db.at[0]], add=True)

    _fire_i(0, dstb0, sem_i0)
    _fire_i(1, dstb1, sem_i1)
    _fire_g(0, rows_a, sem_g0)

    nhalf = _ERPT // 2

    @pl.loop(0, nhalf)
    def _(jj):
        j0 = 2 * jj
        _fire_g(j0 + 1, rows_b, sem_g1)
        _drain_g(rows_a, sem_g0)
        _drain_i(dstb0, sem_i0)
        _work(rows_a, dstb0)

        @pl.when(jj < nhalf - 1)
        def _():
            _fire_i(j0 + 2, dstb0, sem_i0)
            _fire_g(j0 + 2, rows_a, sem_g0)
        _drain_g(rows_b, sem_g1)
        _drain_i(dstb1, sem_i1)
        _work(rows_b, dstb1)

        @pl.when(jj < nhalf - 1)
        def _():
            _fire_i(j0 + 3, dstb1, sem_i1)

    @pl.when(s < _ERTAIL)
    def _():
        _fire_i(_NTILES * _ERPT + s - e0, dstb0, sem_i0)
        _fire_g(_ERPT, rows_a, sem_g0)
        _drain_g(rows_a, sem_g0)
        _drain_i(dstb0, sem_i0)
        _work(rows_a, dstb0)

    plsc.subcore_barrier()

    for j in range(_RBPT):
        blk = s + j * _NTILES
        @pl.when(blk < _NRB)
        def _():
            r0 = pl.multiple_of(blk * _RCH, 8)
            g0 = pl.multiple_of(coff + r0, 8)
            pltpu.sync_copy(acc_s.at[pl.ds(r0, _RCH)], rows_a.at[pl.ds(0, _RCH)])
            pltpu.sync_copy(acc_q.at[pl.ds(r0, _RCH)], sq.at[pl.ds(0, _RCH)])
            pltpu.sync_copy(x_hbm.at[pl.ds(g0, _RCH)], rows_b.at[pl.ds(0, _RCH)])

            @pl.loop(0, _RCH)
            def _(r):
                for k in range(_HALF // 16):
                    sl = pl.ds(k * 16, 16)
                    sm = rows_a[r, sl]
                    rows_a[r, sl] = sm * sm - sq[r, sl] + rows_b[r, sl]

            pltpu.sync_copy(rows_a.at[pl.ds(0, _RCH)], h_hbm.at[pl.ds(g0, _RCH)])


@functools.partial(
    pl.kernel,
    out_type=jax.ShapeDtypeStruct((_B,), jnp.float32),
    mesh=_SC_MESH,
    scratch_types=[
        pltpu.VMEM((2, _CH), jnp.int32),
        pltpu.VMEM((2, _CH), jnp.int32),
        pltpu.VMEM((2, _CH), jnp.int32),
        pltpu.VMEM((2, _CH, _DIM), jnp.float32),
        pltpu.VMEM((2, _CH, _HALF), jnp.float32),
        pltpu.VMEM((2, _CH, _HALF), jnp.float32),
        pltpu.VMEM((_CH,), jnp.float32),
        pltpu.SemaphoreType.DMA,
        pltpu.SemaphoreType.DMA,
    ],
    compiler_params=_SC_PARAMS,
)
def _k3_predict(ut_hbm, h_hbm, u_hbm, i_hbm, out_hbm, ub, ib, ib2, ur, il, ih,
                ob, sem_a, sem_b):
    c = lax.axis_index("c")
    s = lax.axis_index("s")
    wid = s * _NCORES + c
    lanes = lax.broadcasted_iota(jnp.int32, (16,), 0)
    sems = (sem_a, sem_b)
    nch = _BPW // _CH

    def _fetch(ch):
        p = ch % 2
        base = pl.multiple_of(wid * _BPW + ch * _CH, 8)
        pltpu.sync_copy(u_hbm.at[pl.ds(base, _CH)], ub.at[p])
        pltpu.sync_copy(i_hbm.at[pl.ds(base, _CH)], ib.at[p])
        for k in range(_CH // 16):
            sl = pl.ds(k * 16, 16)
            ib2[p, sl] = ib[p, sl] + _N
        pltpu.async_copy(ut_hbm.at[ub.at[p]], ur.at[p], sems[p])
        pltpu.async_copy(h_hbm.at[ib.at[p]], il.at[p], sems[p])
        pltpu.async_copy(h_hbm.at[ib2.at[p]], ih.at[p], sems[p])

    def _drain(ch):
        p = ch % 2
        pltpu.make_async_copy(ut_hbm.at[ub.at[p]], ur.at[p], sems[p]).wait()
        pltpu.make_async_copy(h_hbm.at[ib.at[p]], il.at[p], sems[p]).wait()
        pltpu.make_async_copy(h_hbm.at[ib.at[p]], ih.at[p], sems[p]).wait()

    _fetch(0)
    for ch in range(nch):
        if ch + 1 < nch:
            _fetch(ch + 1)
        _drain(ch)
        p = ch % 2
        base = pl.multiple_of(wid * _BPW + ch * _CH, 8)

        @pl.loop(0, _CH // 16)
        def _(g):
            dot_v = jnp.zeros((16,), jnp.float32)
            ss_v = jnp.zeros((16,), jnp.float32)
            for t in range(16):
                b = g * 16 + t
                acc = jnp.zeros((16,), jnp.float32)
                sacc = jnp.zeros((16,), jnp.float32)
                for k in range(_DIM // 16):
                    uvk = ur[p, b, pl.ds(k * 16, 16)]
                    if k < _HALF // 16:
                        ivk = il[p, b, pl.ds(k * 16, 16)]
                    else:
                        ivk = ih[p, b, pl.ds((k - _HALF // 16) * 16, 16)]
                    acc = acc + uvk * ivk
                    sacc = sacc + uvk * uvk
                dot_v = jnp.where(lanes == t, jnp.sum(acc), dot_v)
                ss_v = jnp.where(lanes == t, jnp.sum(sacc), ss_v)
            y = plsc.bitcast(jnp.int32(0x5F3759DF) - (plsc.bitcast(ss_v, jnp.int32) >> 1),
                             jnp.float32)
            for _ in range(3):
                y = y * (1.5 - 0.5 * ss_v * y * y)
            uvdot = dot_v * jnp.minimum(1.0, y)
            ob[pl.ds(g * 16, 16)] = 1.0 / (1.0 + jnp.exp(-uvdot))

        pltpu.sync_copy(ob, out_hbm.at[pl.ds(base, _CH)])


def kernel(user_table, entity_table, u, i, edge_index):
    u = u.astype(jnp.int32)
    i = i.astype(jnp.int32)
    src = edge_index[0].astype(jnp.int32).reshape(_ER, _CH)
    dst = edge_index[1].astype(jnp.int32).reshape(_ER, _CH)
    x_cat = _maxnorm_split(entity_table)
    h_cat = _k2_aggregate(x_cat, src, dst)
    return _k3_predict(user_table, h_cat, u, i)


# speedup vs baseline: 2.2526x; 1.0292x over previous
import functools

import jax
import jax.numpy as jnp
from jax import lax
from jax.experimental import pallas as pl
from jax.experimental.pallas import tpu as pltpu
from jax.experimental.pallas import tpu_sc as plsc

_N = 10000
_DIM = 128
_HALF = 64
_E = 320000
_B = 16384
_NTILES = 16
_NCORES = 2
_CH = 128
_ER = _E // _CH
_ERPT = _ER // _NTILES
_ERTAIL = _ER - _ERPT * _NTILES
_RCH = 80
_NRB = _N // _RCH
_RBPT = -(-_NRB // _NTILES)
_ACC_ROWS = _N + 16
_BPW = _B // (_NTILES * _NCORES)


def _maxnorm_tc_kernel(x_ref, o_ref):
    x = x_ref[...]
    ss = jnp.sum(x * x, axis=1, keepdims=True)
    n = jnp.sqrt(ss)
    scale = jnp.minimum(1.0, 1.0 / jnp.maximum(n, 1e-7))
    y = x * scale
    o_ref[0] = y[:, :_HALF]
    o_ref[1] = y[:, _HALF:]


def _maxnorm_split(entity_table):
    blk = 1000
    out = pl.pallas_call(
        _maxnorm_tc_kernel,
        grid=(_N // blk,),
        in_specs=[pl.BlockSpec((blk, _DIM), lambda g: (g, 0))],
        out_specs=pl.BlockSpec((2, blk, _HALF), lambda g: (0, g, 0)),
        out_shape=jax.ShapeDtypeStruct((2, _N, _HALF), jnp.float32),
    )(entity_table)
    return out.reshape(2 * _N, _HALF)


_SC_MESH = plsc.VectorSubcoreMesh(core_axis_name="c", subcore_axis_name="s")
_SC_PARAMS = pltpu.CompilerParams(use_tc_tiling_on_sc=False,
                                  needs_layout_passes=False)


@functools.partial(
    pl.kernel,
    out_type=jax.ShapeDtypeStruct((2 * _N, _HALF), jnp.float32),
    mesh=_SC_MESH,
    scratch_types=[
        pltpu.VMEM_SHARED((_ACC_ROWS, _HALF), jnp.float32),
        pltpu.VMEM_SHARED((_ACC_ROWS, _HALF), jnp.float32),
        pltpu.VMEM((_ERPT + 1, _CH), jnp.int32),
        pltpu.VMEM((1, _CH), jnp.int32),
        pltpu.VMEM((1, _CH), jnp.int32),
        pltpu.VMEM((_CH, _HALF), jnp.float32),
        pltpu.VMEM((_CH, _HALF), jnp.float32),
        pltpu.VMEM((_CH, _HALF), jnp.float32),
        pltpu.SemaphoreType.DMA,
        pltpu.SemaphoreType.DMA,
        pltpu.SemaphoreType.DMA,
        pltpu.SemaphoreType.DMA,
        pltpu.SemaphoreType.DMA,
    ],
    compiler_params=_SC_PARAMS,
)
def _k2_aggregate(x_hbm, src_hbm, dst_hbm, h_hbm, acc_s, acc_q, srcb,
                  dstb0, dstb1, rows_a, rows_b, sq,
                  sem_g0, sem_g1, sem_i0, sem_i1, sem_s):
    c = lax.axis_index("c")
    s = lax.axis_index("s")
    coff = c * _N

    e0 = s * _ERPT
    pltpu.sync_copy(src_hbm.at[pl.ds(e0, _ERPT)], srcb.at[pl.ds(0, _ERPT)])

    @pl.when(s < _ERTAIL)
    def _():
        et = _NTILES * _ERPT + s
        pltpu.sync_copy(src_hbm.at[pl.ds(et, 1)], srcb.at[pl.ds(_ERPT, 1)])

    @pl.loop(0, _ERPT + 1)
    def _(r):
        for k in range(_CH // 16):
            sl = pl.ds(k * 16, 16)
            srcb[r, sl] = srcb[r, sl] + coff

    @pl.loop(0, _RCH)
    def _(r):
        for k in range(_HALF // 16):
            rows_a[r, pl.ds(k * 16, 16)] = jnp.zeros((16,), jnp.float32)

    for j in range(_RBPT):
        blk = s + j * _NTILES
        @pl.when(blk < _NRB)
        def _():
            r0 = pl.multiple_of(blk * _RCH, 8)
            pltpu.sync_copy(rows_a.at[pl.ds(0, _RCH)], acc_s.at[pl.ds(r0, _RCH)])
            pltpu.sync_copy(rows_a.at[pl.ds(0, _RCH)], acc_q.at[pl.ds(r0, _RCH)])
    plsc.subcore_barrier()

    def _fire_g(j, rb, sem):
        pltpu.async_copy(x_hbm.at[srcb.at[j]], rb, sem)

    def _drain_g(rb, sem):
        pltpu.make_async_copy(x_hbm.at[srcb.at[0]], rb, sem).wait()

    def _fire_i(j, db, sem):
        pltpu.async_copy(dst_hbm.at[pl.ds(e0 + j, 1)], db, sem)

    def _drain_i(db, sem):
        pltpu.make_async_copy(dst_hbm.at[pl.ds(0, 1)], db, sem).wait()

    def _work(rb, db):
        @pl.loop(0, _CH // 8)
        def _(r8):
            r0 = r8 * 8
            for dr in range(8):
                for k in range(_HALF // 16):
                    sl = pl.ds(k * 16, 16)
                    v = rb[r0 + dr, sl]
                    sq[r0 + dr, sl] = v * v
        pltpu.async_copy(rb, acc_s.at[db.at[0]], sem_s, add=True)
        pltpu.async_copy(sq, acc_q.at[db.at[0]], sem_s, add=True)
        pltpu.make_async_copy(rb, acc_s.at[db.at[0]], sem_s).wait()
        pltpu.make_async_copy(sq, acc_q.at[db.at[0]], sem_s).wait()

    _fire_i(0, dstb0, sem_i0)
    _fire_i(1, dstb1, sem_i1)
    _fire_g(0, rows_a, sem_g0)

    nhalf = _ERPT // 2

    @pl.loop(0, nhalf)
    def _(jj):
        j0 = 2 * jj
        _fire_g(j0 + 1, rows_b, sem_g1)
        _drain_g(rows_a, sem_g0)
        _drain_i(dstb0, sem_i0)
        _work(rows_a, dstb0)

        @pl.when(jj < nhalf - 1)
        def _():
            _fire_i(j0 + 2, dstb0, sem_i0)
            _fire_g(j0 + 2, rows_a, sem_g0)
        _drain_g(rows_b, sem_g1)
        _drain_i(dstb1, sem_i1)
        _work(rows_b, dstb1)

        @pl.when(jj < nhalf - 1)
        def _():
            _fire_i(j0 + 3, dstb1, sem_i1)

    @pl.when(s < _ERTAIL)
    def _():
        _fire_i(_NTILES * _ERPT + s - e0, dstb0, sem_i0)
        _fire_g(_ERPT, rows_a, sem_g0)
        _drain_g(rows_a, sem_g0)
        _drain_i(dstb0, sem_i0)
        _work(rows_a, dstb0)

    plsc.subcore_barrier()

    for j in range(_RBPT):
        blk = s + j * _NTILES
        @pl.when(blk < _NRB)
        def _():
            r0 = pl.multiple_of(blk * _RCH, 8)
            g0 = pl.multiple_of(coff + r0, 8)
            pltpu.sync_copy(acc_s.at[pl.ds(r0, _RCH)], rows_a.at[pl.ds(0, _RCH)])
            pltpu.sync_copy(acc_q.at[pl.ds(r0, _RCH)], sq.at[pl.ds(0, _RCH)])
            pltpu.sync_copy(x_hbm.at[pl.ds(g0, _RCH)], rows_b.at[pl.ds(0, _RCH)])

            @pl.loop(0, _RCH)
            def _(r):
                for k in range(_HALF // 16):
                    sl = pl.ds(k * 16, 16)
                    sm = rows_a[r, sl]
                    rows_a[r, sl] = sm * sm - sq[r, sl] + rows_b[r, sl]

            pltpu.sync_copy(rows_a.at[pl.ds(0, _RCH)], h_hbm.at[pl.ds(g0, _RCH)])


@functools.partial(
    pl.kernel,
    out_type=jax.ShapeDtypeStruct((_B,), jnp.float32),
    mesh=_SC_MESH,
    scratch_types=[
        pltpu.VMEM((2, _CH), jnp.int32),
        pltpu.VMEM((2, _CH), jnp.int32),
        pltpu.VMEM((2, _CH), jnp.int32),
        pltpu.VMEM((2, _CH, _DIM), jnp.float32),
        pltpu.VMEM((2, _CH, _HALF), jnp.float32),
        pltpu.VMEM((2, _CH, _HALF), jnp.float32),
        pltpu.VMEM((_CH,), jnp.float32),
        pltpu.SemaphoreType.DMA,
        pltpu.SemaphoreType.DMA,
    ],
    compiler_params=_SC_PARAMS,
)
def _k3_predict(ut_hbm, h_hbm, u_hbm, i_hbm, out_hbm, ub, ib, ib2, ur, il, ih,
                ob, sem_a, sem_b):
    c = lax.axis_index("c")
    s = lax.axis_index("s")
    wid = s * _NCORES + c
    lanes = lax.broadcasted_iota(jnp.int32, (16,), 0)
    sems = (sem_a, sem_b)
    nch = _BPW // _CH

    def _fetch(ch):
        p = ch % 2
        base = pl.multiple_of(wid * _BPW + ch * _CH, 8)
        pltpu.sync_copy(u_hbm.at[pl.ds(base, _CH)], ub.at[p])
        pltpu.sync_copy(i_hbm.at[pl.ds(base, _CH)], ib.at[p])
        for k in range(_CH // 16):
            sl = pl.ds(k * 16, 16)
            ib2[p, sl] = ib[p, sl] + _N
        pltpu.async_copy(ut_hbm.at[ub.at[p]], ur.at[p], sems[p])
        pltpu.async_copy(h_hbm.at[ib.at[p]], il.at[p], sems[p])
        pltpu.async_copy(h_hbm.at[ib2.at[p]], ih.at[p], sems[p])

    def _drain(ch):
        p = ch % 2
        pltpu.make_async_copy(ut_hbm.at[ub.at[p]], ur.at[p], sems[p]).wait()
        pltpu.make_async_copy(h_hbm.at[ib.at[p]], il.at[p], sems[p]).wait()
        pltpu.make_async_copy(h_hbm.at[ib.at[p]], ih.at[p], sems[p]).wait()

    _fetch(0)
    for ch in range(nch):
        if ch + 1 < nch:
            _fetch(ch + 1)
        _drain(ch)
        p = ch % 2
        base = pl.multiple_of(wid * _BPW + ch * _CH, 8)

        @pl.loop(0, _CH // 16)
        def _(g):
            dot_v = jnp.zeros((16,), jnp.float32)
            ss_v = jnp.zeros((16,), jnp.float32)
            for t in range(16):
                b = g * 16 + t
                acc = jnp.zeros((16,), jnp.float32)
                sacc = jnp.zeros((16,), jnp.float32)
                for k in range(_DIM // 16):
                    uvk = ur[p, b, pl.ds(k * 16, 16)]
                    if k < _HALF // 16:
                        ivk = il[p, b, pl.ds(k * 16, 16)]
                    else:
                        ivk = ih[p, b, pl.ds((k - _HALF // 16) * 16, 16)]
                    acc = acc + uvk * ivk
                    sacc = sacc + uvk * uvk
                dot_v = jnp.where(lanes == t, jnp.sum(acc), dot_v)
                ss_v = jnp.where(lanes == t, jnp.sum(sacc), ss_v)
            y = plsc.bitcast(jnp.int32(0x5F3759DF) - (plsc.bitcast(ss_v, jnp.int32) >> 1),
                             jnp.float32)
            for _ in range(3):
                y = y * (1.5 - 0.5 * ss_v * y * y)
            uvdot = dot_v * jnp.minimum(1.0, y)
            ob[pl.ds(g * 16, 16)] = 1.0 / (1.0 + jnp.exp(-uvdot))

        pltpu.sync_copy(ob, out_hbm.at[pl.ds(base, _CH)])


def kernel(user_table, entity_table, u, i, edge_index):
    u = u.astype(jnp.int32)
    i = i.astype(jnp.int32)
    src = edge_index[0].astype(jnp.int32).reshape(_ER, _CH)
    dst = edge_index[1].astype(jnp.int32).reshape(_ER, _CH)
    x_cat = _maxnorm_split(entity_table)
    h_cat = _k2_aggregate(x_cat, src, dst)
    return _k3_predict(user_table, h_cat, u, i)
